# Initial kernel scaffold; baseline (speedup 1.0000x reference)
import jax
import jax.numpy as jnp
from jax.experimental import pallas as pl


def kernel(labels, weight):
    raise NotImplementedError("write your pallas kernel here")


# speedup vs baseline: 3.0539x; 3.0539x over previous
import functools

import jax
import jax.numpy as jnp
from jax import lax
from jax.experimental import pallas as pl
from jax.experimental.pallas import tpu as pltpu
from jax.experimental.pallas import tpu_sc as plsc

NUM_EMBEDDINGS = 100000
EMBEDDING_DIM = 128
BATCH = 16384
HIST = 50

NC = 2
NS = 16
NW = NC * NS

B_TOTAL = BATCH * HIST
B_PER_W = B_TOTAL // NW
CHUNK = 128
NCHUNK = B_PER_W // CHUNK

_mesh = plsc.VectorSubcoreMesh(
    core_axis_name="c", subcore_axis_name="s", num_cores=NC, num_subcores=NS
)


@functools.partial(
    pl.kernel,
    out_type=jax.ShapeDtypeStruct((B_TOTAL, EMBEDDING_DIM), jnp.float32),
    mesh=_mesh,
    scratch_types=[
        pltpu.VMEM((NCHUNK, CHUNK), jnp.int32),
        pltpu.VMEM((CHUNK, EMBEDDING_DIM), jnp.float32),
        pltpu.SemaphoreType.DMA,
    ],
)
def _sc_gather(idx_hbm, table_hbm, out_hbm, idx_v, rows_v, sem):
    wid = lax.axis_index("s") * NC + lax.axis_index("c")
    base = wid * B_PER_W
    pltpu.sync_copy(idx_hbm.at[wid], idx_v)

    @pl.loop(0, NCHUNK)
    def _chunk(j):
        pltpu.async_copy(table_hbm.at[idx_v.at[j]], rows_v, sem).wait()
        pltpu.sync_copy(rows_v, out_hbm.at[pl.ds(base + j * CHUNK, CHUNK)])


def kernel(labels, weight):
    idx = labels.reshape(NW, NCHUNK, CHUNK)
    out = _sc_gather(idx, weight)
    return out.reshape(BATCH, HIST, EMBEDDING_DIM)


# speedup vs baseline: 3.4680x; 1.1356x over previous
import functools

import jax
import jax.numpy as jnp
from jax import lax
from jax.experimental import pallas as pl
from jax.experimental.pallas import tpu as pltpu
from jax.experimental.pallas import tpu_sc as plsc

NUM_EMBEDDINGS = 100000
EMBEDDING_DIM = 128
BATCH = 16384
HIST = 50

NC = 2
NS = 16
NW = NC * NS

B_TOTAL = BATCH * HIST
B_PER_W = B_TOTAL // NW
CHUNK = 128
NCHUNK = B_PER_W // CHUNK
NBUF = 4

_mesh = plsc.VectorSubcoreMesh(
    core_axis_name="c", subcore_axis_name="s", num_cores=NC, num_subcores=NS
)


@functools.partial(
    pl.kernel,
    out_type=jax.ShapeDtypeStruct((B_TOTAL, EMBEDDING_DIM), jnp.float32),
    mesh=_mesh,
    scratch_types=[
        pltpu.VMEM((NCHUNK, CHUNK), jnp.int32),
        [pltpu.VMEM((CHUNK, EMBEDDING_DIM), jnp.float32) for _ in range(NBUF)],
        [pltpu.SemaphoreType.DMA for _ in range(NBUF)],
        [pltpu.SemaphoreType.DMA for _ in range(NBUF)],
    ],
)
def _sc_gather(idx_hbm, table_hbm, out_hbm, idx_v, rows, gsem, wsem):
    wid = lax.axis_index("s") * NC + lax.axis_index("c")
    base = wid * B_PER_W
    pltpu.sync_copy(idx_hbm.at[wid], idx_v)

    for b in range(NBUF):
        pltpu.async_copy(table_hbm.at[idx_v.at[b]], rows[b], gsem[b])

    @pl.loop(0, NCHUNK, step=NBUF)
    def _group(j):
        for b in range(NBUF):
            pltpu.make_async_copy(
                table_hbm.at[idx_v.at[j + b]], rows[b], gsem[b]
            ).wait()
            pltpu.async_copy(
                rows[b], out_hbm.at[pl.ds(base + (j + b) * CHUNK, CHUNK)],
                wsem[b],
            )
            nxt = j + b + NBUF

            @pl.when(nxt < NCHUNK)
            def _refill():
                pltpu.make_async_copy(
                    rows[b],
                    out_hbm.at[pl.ds(base + (j + b) * CHUNK, CHUNK)],
                    wsem[b],
                ).wait()
                pltpu.async_copy(table_hbm.at[idx_v.at[nxt]], rows[b], gsem[b])

    for b in range(NBUF):
        j_last = NCHUNK - NBUF + b
        pltpu.make_async_copy(
            rows[b], out_hbm.at[pl.ds(base + j_last * CHUNK, CHUNK)], wsem[b]
        ).wait()


def kernel(labels, weight):
    idx = labels.reshape(NW, NCHUNK, CHUNK)
    out = _sc_gather(idx, weight)
    return out.reshape(BATCH, HIST, EMBEDDING_DIM)
